# baseline (speedup 1.0000x reference)
import functools

import jax
import jax.numpy as jnp
from jax.experimental import pallas as pl


def _tree_lstm_block(x_ref, mh_ref, mc_ref,
                     Wfh_t_ref, bf_h_ref, Ufh_t_ref,
                     Wiou_t_ref, biou_ref, Uiou_t_ref,
                     h_ref, c_ref, *, k: int, h_dim: int):
    xb = x_ref[...]

    wx_h = jnp.dot(xb, Wfh_t_ref[...],
                   preferred_element_type=jnp.float32) + bf_h_ref[...]

    h_acc = jnp.zeros_like(wx_h)
    mc_acc = jnp.zeros_like(wx_h)
    t_acc = jnp.zeros_like(wx_h)
    Ufh_t = Ufh_t_ref[...]
    for ki in range(k):
        mh_k = mh_ref[:, ki, :]
        mc_k = mc_ref[:, ki, :]
        z_h = jnp.dot(mh_k, Ufh_t, preferred_element_type=jnp.float32) + wx_h
        t_acc = t_acc + jnp.tanh(z_h) * mc_k
        mc_acc = mc_acc + mc_k
        h_acc = h_acc + mh_k

    c_tild = 0.5 * (t_acc + mc_acc)

    iou = (jnp.dot(xb, Wiou_t_ref[...], preferred_element_type=jnp.float32)
           + jnp.dot(h_acc, Uiou_t_ref[...],
                     preferred_element_type=jnp.float32)
           + biou_ref[...])
    i_g = jax.nn.sigmoid(iou[:, :h_dim])
    o_g = jax.nn.sigmoid(iou[:, h_dim:2 * h_dim])
    u_g = jnp.tanh(iou[:, 2 * h_dim:])

    c = i_g * u_g + c_tild
    h_ref[...] = o_g * jnp.tanh(c)
    c_ref[...] = c


def kernel(x, msgs_h, msgs_c, W_iou, b_iou, U_iou, b_uiou, W_f, b_wf, U_f, b_uf):
    n, k, h_dim = msgs_h.shape
    x_dim = x.shape[1]

    block_n = 400
    assert n % block_n == 0
    grid = (n // block_n,)

    full = lambda i: (0, 0)
    body = functools.partial(_tree_lstm_block, k=k, h_dim=h_dim)

    h, c = pl.pallas_call(
        body,
        grid=grid,
        in_specs=[
            pl.BlockSpec((block_n, x_dim), lambda i: (i, 0)),
            pl.BlockSpec((block_n, k, h_dim), lambda i: (i, 0, 0)),
            pl.BlockSpec((block_n, k, h_dim), lambda i: (i, 0, 0)),
            pl.BlockSpec((x_dim, h_dim), full),
            pl.BlockSpec((1, h_dim), full),
            pl.BlockSpec((h_dim, h_dim), full),
            pl.BlockSpec((x_dim, 3 * h_dim), full),
            pl.BlockSpec((1, 3 * h_dim), full),
            pl.BlockSpec((h_dim, 3 * h_dim), full),
        ],
        out_specs=[
            pl.BlockSpec((block_n, h_dim), lambda i: (i, 0)),
            pl.BlockSpec((block_n, h_dim), lambda i: (i, 0)),
        ],
        out_shape=[
            jax.ShapeDtypeStruct((n, h_dim), jnp.float32),
            jax.ShapeDtypeStruct((n, h_dim), jnp.float32),
        ],
    )(
        x, msgs_h, msgs_c,
        0.5 * W_f.T, (0.5 * (b_wf + b_uf)).reshape(1, h_dim),
        0.5 * U_f.T,
        W_iou.T, (b_iou + b_uiou).reshape(1, 3 * h_dim),
        U_iou.T,
    )
    return (h, c)


# speedup vs baseline: 3.1547x; 3.1547x over previous
import functools

import jax
import jax.numpy as jnp
from jax.experimental import pallas as pl


def _tree_lstm_block(x_ref, mh_ref, mc_ref,
                     Wfh_t_ref, bfh_ref, Ufh_t_ref,
                     Wiou_t_ref, biou_ref, Uiou_t_ref,
                     h_ref, c_ref, *, block_n: int, k: int, h_dim: int):
    xb = x_ref[...]
    mh = mh_ref[...]
    mc = mc_ref[...]

    h_tild = jnp.sum(mh, axis=1)
    mc_sum = jnp.sum(mc, axis=1)

    wx_h = jnp.dot(xb, Wfh_t_ref[...],
                   preferred_element_type=jnp.float32) + bfh_ref[...]

    mh2 = mh.reshape(block_n * k, h_dim)
    uh_h = jnp.dot(mh2, Ufh_t_ref[...], preferred_element_type=jnp.float32)
    t = jnp.tanh(uh_h.reshape(block_n, k, h_dim) + wx_h[:, None, :])
    c_tild = 0.5 * (jnp.sum(t * mc, axis=1) + mc_sum)

    iou = (jnp.dot(xb, Wiou_t_ref[...], preferred_element_type=jnp.float32)
           + jnp.dot(h_tild, Uiou_t_ref[...],
                     preferred_element_type=jnp.float32)
           + biou_ref[...])
    i_g = 0.5 * jnp.tanh(iou[:, :h_dim]) + 0.5
    o_g = 0.5 * jnp.tanh(iou[:, h_dim:2 * h_dim]) + 0.5
    u_g = jnp.tanh(iou[:, 2 * h_dim:])

    c = i_g * u_g + c_tild
    h_ref[...] = o_g * jnp.tanh(c)
    c_ref[...] = c


def kernel(x, msgs_h, msgs_c, W_iou, b_iou, U_iou, b_uiou, W_f, b_wf, U_f, b_uf):
    n, k, h_dim = msgs_h.shape
    x_dim = x.shape[1]

    block_n = 400
    assert n % block_n == 0
    grid = (n // block_n,)

    iou_scale = jnp.concatenate(
        [jnp.full((2 * h_dim,), 0.5, jnp.float32),
         jnp.ones((h_dim,), jnp.float32)])

    full = lambda i: (0, 0)
    body = functools.partial(_tree_lstm_block, block_n=block_n, k=k,
                             h_dim=h_dim)

    h, c = pl.pallas_call(
        body,
        grid=grid,
        in_specs=[
            pl.BlockSpec((block_n, x_dim), lambda i: (i, 0)),
            pl.BlockSpec((block_n, k, h_dim), lambda i: (i, 0, 0)),
            pl.BlockSpec((block_n, k, h_dim), lambda i: (i, 0, 0)),
            pl.BlockSpec((x_dim, h_dim), full),
            pl.BlockSpec((1, h_dim), full),
            pl.BlockSpec((h_dim, h_dim), full),
            pl.BlockSpec((x_dim, 3 * h_dim), full),
            pl.BlockSpec((1, 3 * h_dim), full),
            pl.BlockSpec((h_dim, 3 * h_dim), full),
        ],
        out_specs=[
            pl.BlockSpec((block_n, h_dim), lambda i: (i, 0)),
            pl.BlockSpec((block_n, h_dim), lambda i: (i, 0)),
        ],
        out_shape=[
            jax.ShapeDtypeStruct((n, h_dim), jnp.float32),
            jax.ShapeDtypeStruct((n, h_dim), jnp.float32),
        ],
    )(
        x, msgs_h, msgs_c,
        0.5 * W_f.T, (0.5 * (b_wf + b_uf)).reshape(1, h_dim),
        0.5 * U_f.T,
        W_iou.T * iou_scale, ((b_iou + b_uiou) * iou_scale).reshape(1, 3 * h_dim),
        U_iou.T * iou_scale,
    )
    return (h, c)
